# Initial kernel scaffold; baseline (speedup 1.0000x reference)
import jax
import jax.numpy as jnp
from jax.experimental import pallas as pl


def kernel(input, idxn, degs):
    raise NotImplementedError("write your pallas kernel here")


# speedup vs baseline: 8.6225x; 8.6225x over previous
import functools

import jax
import jax.numpy as jnp
from jax import lax
from jax.experimental import pallas as pl
from jax.experimental.pallas import tpu as pltpu
from jax.experimental.pallas import tpu_sc as plsc

N_NODES = 10000
N_EDGES = 320000
N_POOLED = 10000
D_FEAT = 128
DEG = 32

NC = 2
NS = 16
NW = NC * NS

SW = 320
M_PAD = NW * SW
CS = 4
ROWS = CS * DEG
NCH = SW // CS
LANES = 16
NV = D_FEAT // LANES
INV_DEG = 1.0 / DEG


def _tree_sum(vals):
    while len(vals) > 1:
        nxt = [vals[i] + vals[i + 1] for i in range(0, len(vals) - 1, 2)]
        if len(vals) % 2:
            nxt.append(vals[-1])
        vals = nxt
    return vals[0]


def _sc_body(table_hbm, idx_hbm, out_hbm, idx_v, rows0, rows1, out_v,
             sem0, sem1):
    wid = lax.axis_index("s") * NC + lax.axis_index("c")
    seg_base = wid * SW

    pltpu.sync_copy(idx_hbm.at[wid], idx_v)

    def gather(ch, buf, sem):
        return pltpu.async_copy(table_hbm.at[idx_v.at[ch]], buf, sem)

    def compute(rows_ref, ch):
        for s in range(CS):
            for d in range(NV):
                col = pl.ds(d * LANES, LANES)
                acc = _tree_sum([rows_ref[s * DEG + r, col]
                                 for r in range(DEG)])
                out_v[ch * CS + s, col] = acc * INV_DEG

    gather(0, rows0, sem0)
    gather(1, rows1, sem1)

    def body(ph, carry):
        ch = ph * 2
        pltpu.make_async_copy(table_hbm.at[idx_v.at[ch]], rows0, sem0).wait()
        compute(rows0, ch)

        @pl.when(ch + 2 < NCH)
        def _():
            gather(ch + 2, rows0, sem0)

        pltpu.make_async_copy(
            table_hbm.at[idx_v.at[ch + 1]], rows1, sem1).wait()
        compute(rows1, ch + 1)

        @pl.when(ch + 3 < NCH)
        def _():
            gather(ch + 3, rows1, sem1)

        return carry

    lax.fori_loop(0, NCH // 2, body, 0)

    pltpu.sync_copy(out_v, out_hbm.at[pl.ds(seg_base, SW)])


@jax.jit
def _pooled_mean(table, idx_blocks):
    mesh = plsc.VectorSubcoreMesh(core_axis_name="c", subcore_axis_name="s")
    return pl.kernel(
        _sc_body,
        out_type=jax.ShapeDtypeStruct((M_PAD, D_FEAT), jnp.float32),
        mesh=mesh,
        scratch_types=[
            pltpu.VMEM((NCH, ROWS), jnp.int32),
            pltpu.VMEM((ROWS, D_FEAT), jnp.float32),
            pltpu.VMEM((ROWS, D_FEAT), jnp.float32),
            pltpu.VMEM((SW, D_FEAT), jnp.float32),
            pltpu.SemaphoreType.DMA,
            pltpu.SemaphoreType.DMA,
        ],
    )(table, idx_blocks)


def kernel(input, idxn, degs):
    del degs
    pad = M_PAD * DEG - N_EDGES
    idx_p = jnp.concatenate(
        [idxn, jnp.zeros((pad,), jnp.int32)]).reshape(NW, NCH, ROWS)
    out = _pooled_mean(input, idx_p)
    return out[:N_POOLED]
